# baseline (speedup 1.0000x reference)
import functools
import math

import jax
import jax.numpy as jnp
from jax import lax
from jax.experimental import pallas as pl
from jax.experimental.pallas import tpu as pltpu
from jax.experimental.pallas import tpu_sc as plsc

DIM = 128
SCALE = math.sqrt(float(DIM))

_NC = 2
_NS = 16
_NW = _NC * _NS

CHUNK = 128


@functools.lru_cache(maxsize=None)
def _make_kernel(n_idx):
    assert n_idx % (_NW * CHUNK) == 0
    chunks_per_w = n_idx // (_NW * CHUNK)
    assert chunks_per_w % 2 == 0 and chunks_per_w >= 4
    mesh = plsc.VectorSubcoreMesh(core_axis_name="c", subcore_axis_name="s")

    @functools.partial(
        pl.kernel,
        mesh=mesh,
        out_type=jax.ShapeDtypeStruct((n_idx, DIM), jnp.float32),
        scratch_types=[
            pltpu.VMEM((chunks_per_w, CHUNK), jnp.int32),
            pltpu.VMEM((2, CHUNK, DIM), jnp.float32),
            pltpu.SemaphoreType.DMA,
            pltpu.SemaphoreType.DMA,
            pltpu.SemaphoreType.DMA,
            pltpu.SemaphoreType.DMA,
        ],
    )
    def body(idx_hbm, table_hbm, out_hbm, idx_v, rows_v, g0, g1, o0, o1):
        wid = lax.axis_index("s") * _NC + lax.axis_index("c")
        row0 = wid * chunks_per_w
        pltpu.sync_copy(idx_hbm.at[wid], idx_v)

        gsem = (g0, g1)
        osem = (o0, o1)

        def g_copy(j, b):
            return pltpu.make_async_copy(
                table_hbm.at[idx_v.at[j]], rows_v.at[b], gsem[b])

        def o_copy(j, b):
            return pltpu.make_async_copy(
                rows_v.at[b],
                out_hbm.at[pl.ds((row0 + j) * CHUNK, CHUNK)],
                osem[b])

        def scale(b):
            @plsc.parallel_loop(0, CHUNK, step=1, unroll=4)
            def _sb(i):
                for u in range(DIM // 16):
                    sl = pl.ds(u * 16, 16)
                    rows_v[b, i, sl] = rows_v[b, i, sl] * SCALE

        def step(j, b, first, last):
            g_copy(j, b).wait()
            if not first:
                o_copy(j - 1, 1 - b).wait()
            if not last:
                g_copy(j + 1, 1 - b).start()
            scale(b)
            o_copy(j, b).start()

        g_copy(0, 0).start()
        step(0, 0, True, False)
        step(1, 1, False, False)

        def loop_body(jp, c):
            step(2 * jp, 0, False, False)
            step(2 * jp + 1, 1, False, False)
            return c

        lax.fori_loop(1, chunks_per_w // 2 - 1, loop_body, 0)

        step(chunks_per_w - 2, 0, False, False)
        step(chunks_per_w - 1, 1, False, True)
        o_copy(chunks_per_w - 1, 1).wait()

    return body


def kernel(input, table):
    b, h = input.shape
    idx2 = input.reshape(_NW, b * h // (_NW * CHUNK), CHUNK)
    out = _make_kernel(b * h)(idx2, table)
    return out.reshape(b, h, DIM)


# speedup vs baseline: 1.6356x; 1.6356x over previous
import functools
import math

import jax
import jax.numpy as jnp
from jax import lax
from jax.experimental import pallas as pl
from jax.experimental.pallas import tpu as pltpu
from jax.experimental.pallas import tpu_sc as plsc

DIM = 128
SCALE = math.sqrt(float(DIM))

_NC = 2
_NS = 16
_NW = _NC * _NS


@functools.lru_cache(maxsize=None)
def _make_kernel(batch, hist):
    b_per_w = batch // _NW
    chunk_b = 2
    chunk = chunk_b * hist
    assert batch % _NW == 0 and chunk <= 128
    n_chunks = b_per_w // chunk_b
    assert n_chunks % 2 == 0 and n_chunks >= 4
    mesh = plsc.VectorSubcoreMesh(core_axis_name="c", subcore_axis_name="s")

    @functools.partial(
        pl.kernel,
        mesh=mesh,
        out_type=jax.ShapeDtypeStruct((batch, hist, DIM), jnp.float32),
        scratch_types=[
            pltpu.VMEM((n_chunks, chunk), jnp.int32),
            pltpu.VMEM((2, chunk, DIM), jnp.float32),
            pltpu.SemaphoreType.DMA,
            pltpu.SemaphoreType.DMA,
            pltpu.SemaphoreType.DMA,
            pltpu.SemaphoreType.DMA,
        ],
    )
    def body(idx_hbm, table_hbm, out_hbm, idx_v, rows_v, g0, g1, o0, o1):
        wid = lax.axis_index("s") * _NC + lax.axis_index("c")
        b0 = wid * b_per_w
        pltpu.sync_copy(idx_hbm.at[wid], idx_v)

        gsem = (g0, g1)
        osem = (o0, o1)

        def g_copy(j, b):
            return pltpu.make_async_copy(
                table_hbm.at[idx_v.at[j]], rows_v.at[b], gsem[b])

        def o_copies(j, b):
            bb = b0 + chunk_b * j
            return [
                pltpu.make_async_copy(
                    rows_v.at[b, pl.ds(u * hist, hist)],
                    out_hbm.at[bb + u], osem[b])
                for u in range(chunk_b)
            ]

        def scale(b):
            @plsc.parallel_loop(0, chunk, step=1, unroll=4)
            def _sb(i):
                for u in range(DIM // 16):
                    sl = pl.ds(u * 16, 16)
                    rows_v[b, i, sl] = rows_v[b, i, sl] * SCALE

        def step(j, b, first, last):
            g_copy(j, b).wait()
            if not first:
                for c in o_copies(j - 1, 1 - b):
                    c.wait()
            if not last:
                g_copy(j + 1, 1 - b).start()
            scale(b)
            for c in o_copies(j, b):
                c.start()

        g_copy(0, 0).start()
        step(0, 0, True, False)
        step(1, 1, False, False)

        def loop_body(jp, c):
            step(2 * jp, 0, False, False)
            step(2 * jp + 1, 1, False, False)
            return c

        lax.fori_loop(1, n_chunks // 2 - 1, loop_body, 0)

        step(n_chunks - 2, 0, False, False)
        step(n_chunks - 1, 1, False, True)
        for c in o_copies(n_chunks - 1, 1):
            c.wait()

    return body


def kernel(input, table):
    batch, hist = input.shape
    b_per_w = batch // _NW
    idx3 = input.reshape(_NW, b_per_w // 2, 2 * hist)
    return _make_kernel(batch, hist)(idx3, table)
